# baseline (speedup 1.0000x reference)
import jax
import jax.numpy as jnp
from jax import lax
from jax.experimental import pallas as pl
from jax.experimental.pallas import tpu as pltpu

_N_SLOTS = 10
_B, _C, _H, _W = 4, 384, 32, 32
_RPB = _H * _W
_ROWS = _B * _RPB
_K = 1
_RCHUNK = _ROWS // _K


def _phase1_body(f1_ref, f2_ref, f3_ref, mi_ref, idx_ref, acc_ref):
    k = pl.program_id(0)
    i = pl.program_id(1)

    @pl.when((k == 0) & (i == 0))
    def _init():
        acc_ref[...] = jnp.zeros_like(acc_ref)

    ones = jnp.ones((1, _RCHUNK), jnp.float32)
    for c, f_ref in enumerate((f1_ref, f2_ref, f3_ref)):
        diff = mi_ref[0, c] - f_ref[...]
        s = diff * diff
        part = jax.lax.dot(ones, s,
                           preferred_element_type=jnp.float32)
        j = c * _N_SLOTS + i
        acc_ref[pl.ds(j, 1)] += part.reshape(1, 1, _C)

    @pl.when((k == _K - 1) & (i == _N_SLOTS - 1))
    def _finish():
        pair = jnp.sum(acc_ref[...], axis=2)
        r = jnp.sqrt(pair)
        d = r[0:10] + r[10:20] + r[20:30]
        m = jnp.min(d)
        sub = lax.broadcasted_iota(jnp.int32, (10, 1), 0)
        idx_ref[0, 0] = jnp.min(jnp.where(d == m, sub, 127))


def _phase2_body(idx_ref, f1_ref, f2_ref, f3_ref, mi_ref,
                 ci1_ref, ci2_ref, ci3_ref, sel_ref):
    c = pl.program_id(1)
    mi = mi_ref[0, 0]

    @pl.when(c == 0)
    def _():
        ci1_ref[0, :, :_C] = f1_ref[...]
        ci1_ref[0, :, _C:] = mi

    @pl.when(c == 1)
    def _():
        ci2_ref[0, :, :_C] = f2_ref[...]
        ci2_ref[0, :, _C:] = mi

    @pl.when(c == 2)
    def _():
        ci3_ref[0, :, :_C] = f3_ref[...]
        ci3_ref[0, :, _C:] = mi

    sel_ref[0, 0] = mi


def kernel(feature1, feature2, feature3, MI):
    f1 = feature1.transpose(0, 2, 3, 1).reshape(_ROWS, _C)
    f2 = feature2.transpose(0, 2, 3, 1).reshape(_ROWS, _C)
    f3 = feature3.transpose(0, 2, 3, 1).reshape(_ROWS, _C)
    mi4 = MI.transpose(0, 1, 2, 4, 5, 3).reshape(_N_SLOTS, 3, _ROWS, _C)

    feat_spec = pl.BlockSpec((_RCHUNK, _C), lambda k, i: (k, 0))
    idx = pl.pallas_call(
        _phase1_body,
        grid=(_K, _N_SLOTS),
        in_specs=[
            feat_spec, feat_spec, feat_spec,
            pl.BlockSpec((1, 3, _RCHUNK, _C),
                         lambda k, i: (i, 0, k, 0)),
        ],
        out_specs=pl.BlockSpec(memory_space=pltpu.SMEM),
        out_shape=jax.ShapeDtypeStruct((1, 1), jnp.int32),
        scratch_shapes=[pltpu.VMEM((32, 1, _C), jnp.float32)],
    )(f1, f2, f3, mi4)

    fspec = pl.BlockSpec((_RPB, _C), lambda n, c, idx_ref: (n, 0))
    cspec = pl.BlockSpec((1, _RPB, 2 * _C), lambda n, c, idx_ref: (n, 0, 0))
    grid_spec = pltpu.PrefetchScalarGridSpec(
        num_scalar_prefetch=1,
        grid=(_B, 3),
        in_specs=[
            fspec, fspec, fspec,
            pl.BlockSpec((1, 1, _RPB, _C),
                         lambda n, c, idx_ref: (idx_ref[0], c, n, 0)),
        ],
        out_specs=[
            cspec, cspec, cspec,
            pl.BlockSpec((1, 1, _RPB, _C),
                         lambda n, c, idx_ref: (c, n, 0, 0)),
        ],
    )
    ci1, ci2, ci3, sel = pl.pallas_call(
        _phase2_body,
        grid_spec=grid_spec,
        out_shape=[
            jax.ShapeDtypeStruct((_B, _RPB, 2 * _C), jnp.float32),
            jax.ShapeDtypeStruct((_B, _RPB, 2 * _C), jnp.float32),
            jax.ShapeDtypeStruct((_B, _RPB, 2 * _C), jnp.float32),
            jax.ShapeDtypeStruct((3, _B, _RPB, _C), jnp.float32),
        ],
    )(idx.reshape(1), f1, f2, f3, mi4)

    def _to_nchw(ci):
        return ci.reshape(_B, _H, _W, 2 * _C).transpose(0, 3, 1, 2)

    sel_out = sel.reshape(3, _B, _H, _W, _C).transpose(0, 1, 4, 2, 3)
    return (_to_nchw(ci1), _to_nchw(ci2), _to_nchw(ci3), sel_out)


# speedup vs baseline: 1.0470x; 1.0470x over previous
import jax
import jax.numpy as jnp
from jax import lax
from jax.experimental import pallas as pl
from jax.experimental.pallas import tpu as pltpu

_N_SLOTS = 10
_B, _C, _H, _W = 4, 384, 32, 32
_RPB = _H * _W
_ROWS = _B * _RPB
_K = 2
_RCHUNK = _ROWS // _K


def _phase1_body(f1_ref, f2_ref, f3_ref, mi_ref,
                 idx_ref, c1_ref, c2_ref, c3_ref, acc_ref):
    k = pl.program_id(0)
    i = pl.program_id(1)

    @pl.when((k == 0) & (i == 0))
    def _init():
        acc_ref[...] = jnp.zeros_like(acc_ref)

    @pl.when(i == 0)
    def _feature_halves():
        c1_ref[...] = f1_ref[...]
        c2_ref[...] = f2_ref[...]
        c3_ref[...] = f3_ref[...]

    ones = jnp.ones((1, _RCHUNK), jnp.float32)
    for c, f_ref in enumerate((f1_ref, f2_ref, f3_ref)):
        diff = mi_ref[0, c] - f_ref[...]
        s = diff * diff
        part = jax.lax.dot(ones, s,
                           preferred_element_type=jnp.float32)
        j = c * _N_SLOTS + i
        acc_ref[pl.ds(j, 1)] += part.reshape(1, 1, _C)

    @pl.when((k == _K - 1) & (i == _N_SLOTS - 1))
    def _finish():
        pair = jnp.sum(acc_ref[...], axis=2)
        r = jnp.sqrt(pair)
        d = r[0:10] + r[10:20] + r[20:30]
        m = jnp.min(d)
        sub = lax.broadcasted_iota(jnp.int32, (10, 1), 0)
        idx_ref[0, 0] = jnp.min(jnp.where(d == m, sub, 127))


def _phase2_body(idx_ref, mi_ref, c1in_ref, c2in_ref, c3in_ref,
                 ci1_ref, ci2_ref, ci3_ref, sel_ref):
    c = pl.program_id(1)
    mi = mi_ref[0, 0]

    @pl.when(c == 0)
    def _():
        ci1_ref[0] = mi

    @pl.when(c == 1)
    def _():
        ci2_ref[0] = mi

    @pl.when(c == 2)
    def _():
        ci3_ref[0] = mi

    sel_ref[0, 0] = mi


def kernel(feature1, feature2, feature3, MI):
    f1 = feature1.transpose(0, 2, 3, 1).reshape(_ROWS, _C)
    f2 = feature2.transpose(0, 2, 3, 1).reshape(_ROWS, _C)
    f3 = feature3.transpose(0, 2, 3, 1).reshape(_ROWS, _C)
    mi4 = MI.transpose(0, 1, 2, 4, 5, 3).reshape(_N_SLOTS, 3, _ROWS, _C)

    feat_spec = pl.BlockSpec((_RCHUNK, _C), lambda k, i: (k, 0))
    chalf_spec = pl.BlockSpec((_RCHUNK, _C), lambda k, i: (k, 0))
    ci_shape = jax.ShapeDtypeStruct((_ROWS, 2 * _C), jnp.float32)
    idx, ci1h, ci2h, ci3h = pl.pallas_call(
        _phase1_body,
        grid=(_K, _N_SLOTS),
        in_specs=[
            feat_spec, feat_spec, feat_spec,
            pl.BlockSpec((1, 3, _RCHUNK, _C),
                         lambda k, i: (i, 0, k, 0)),
        ],
        out_specs=[
            pl.BlockSpec(memory_space=pltpu.SMEM),
            chalf_spec, chalf_spec, chalf_spec,
        ],
        out_shape=[
            jax.ShapeDtypeStruct((1, 1), jnp.int32),
            ci_shape, ci_shape, ci_shape,
        ],
        scratch_shapes=[pltpu.VMEM((32, 1, _C), jnp.float32)],
    )(f1, f2, f3, mi4)

    cin_spec = pl.BlockSpec(memory_space=pl.ANY)
    cout_spec = pl.BlockSpec((1, _RPB, _C), lambda n, c, idx_ref: (n, 0, 1))
    grid_spec = pltpu.PrefetchScalarGridSpec(
        num_scalar_prefetch=1,
        grid=(_B, 3),
        in_specs=[
            pl.BlockSpec((1, 1, _RPB, _C),
                         lambda n, c, idx_ref: (idx_ref[0], c, n, 0)),
            cin_spec, cin_spec, cin_spec,
        ],
        out_specs=[
            cout_spec, cout_spec, cout_spec,
            pl.BlockSpec((1, 1, _RPB, _C),
                         lambda n, c, idx_ref: (c, n, 0, 0)),
        ],
    )
    ci1, ci2, ci3, sel = pl.pallas_call(
        _phase2_body,
        grid_spec=grid_spec,
        out_shape=[
            jax.ShapeDtypeStruct((_B, _RPB, 2 * _C), jnp.float32),
            jax.ShapeDtypeStruct((_B, _RPB, 2 * _C), jnp.float32),
            jax.ShapeDtypeStruct((_B, _RPB, 2 * _C), jnp.float32),
            jax.ShapeDtypeStruct((3, _B, _RPB, _C), jnp.float32),
        ],
        input_output_aliases={2: 0, 3: 1, 4: 2},
    )(idx.reshape(1), mi4,
      ci1h.reshape(_B, _RPB, 2 * _C),
      ci2h.reshape(_B, _RPB, 2 * _C),
      ci3h.reshape(_B, _RPB, 2 * _C))

    def _to_nchw(ci):
        return ci.reshape(_B, _H, _W, 2 * _C).transpose(0, 3, 1, 2)

    sel_out = sel.reshape(3, _B, _H, _W, _C).transpose(0, 1, 4, 2, 3)
    return (_to_nchw(ci1), _to_nchw(ci2), _to_nchw(ci3), sel_out)
